# baseline (speedup 1.0000x reference)
import functools

import jax
import jax.numpy as jnp
from jax import lax
from jax.experimental import pallas as pl
from jax.experimental.pallas import tpu as pltpu
from jax.experimental.pallas import tpu_sc as plsc

BATCH = 4096
HIST = 200
D_MODEL = 64
_B = BATCH * HIST

_info = plsc.get_sparse_core_info()
_NC = _info.num_cores
_NS = _info.num_subcores
_NW = _NC * _NS
_BPW = BATCH // _NW
_CH = 100
_NCH = _BPW * HIST // _CH
_NBUF = 4
_ROUNDS = _NCH // _NBUF


def _emb_body(idx_hbm, table_hbm, out_hbm, idx_v, rows_v, *sems):
    gsems = sems[:_NBUF]
    osems = sems[_NBUF:]
    wid = lax.axis_index("s") * _NC + lax.axis_index("c")
    bbase = wid * _BPW
    pltpu.sync_copy(idx_hbm.at[wid], idx_v)

    def g_start(j, b):
        pltpu.async_copy(table_hbm.at[idx_v.at[j]], rows_v.at[b], gsems[b])

    def g_wait(b):
        pltpu.make_async_copy(
            table_hbm.at[idx_v.at[0]], rows_v.at[b], gsems[b]).wait()

    def o_start(j, b):
        bi = bbase + j // 2
        h0 = (j % 2) * _CH
        pltpu.async_copy(
            rows_v.at[b], out_hbm.at[bi, pl.ds(h0, _CH)], osems[b])

    def o_wait(b):
        pltpu.make_async_copy(
            rows_v.at[b], out_hbm.at[0, pl.ds(0, _CH)], osems[b]).wait()

    for b in range(_NBUF):
        g_start(b, b)

    def round_body(r, carry):
        jbase = r * _NBUF
        for b in range(_NBUF):
            g_wait(b)
            o_start(jbase + b, b)
        for b in range(_NBUF):
            o_wait(b)
            g_start(jbase + _NBUF + b, b)
        return carry

    lax.fori_loop(0, _ROUNDS - 1, round_body, 0)

    jlast = (_ROUNDS - 1) * _NBUF
    for b in range(_NBUF):
        g_wait(b)
        o_start(jlast + b, b)
    for b in range(_NBUF):
        o_wait(b)


@jax.jit
def _embed(idx3d, table):
    mesh = plsc.VectorSubcoreMesh(core_axis_name="c", subcore_axis_name="s")
    k = functools.partial(
        pl.kernel,
        mesh=mesh,
        out_type=jax.ShapeDtypeStruct((BATCH, HIST, D_MODEL), jnp.float32),
        scratch_types=[
            pltpu.VMEM((_NCH, _CH), jnp.int32),
            pltpu.VMEM((_NBUF, _CH, D_MODEL), jnp.float32),
        ] + [pltpu.SemaphoreType.DMA] * (2 * _NBUF),
        compiler_params=pltpu.CompilerParams(use_tc_tiling_on_sc=False),
    )(_emb_body)
    return k(idx3d, table)


def kernel(input_ids, embedding_weight):
    idx = input_ids.reshape(-1).astype(jnp.int32)
    idx3d = idx.reshape(_NW, _NCH, _CH)
    return _embed(idx3d, embedding_weight)


# speedup vs baseline: 1.3183x; 1.3183x over previous
import functools

import jax
import jax.numpy as jnp
from jax import lax
from jax.experimental import pallas as pl
from jax.experimental.pallas import tpu as pltpu
from jax.experimental.pallas import tpu_sc as plsc

BATCH = 4096
HIST = 200
D_MODEL = 64
_DPAD = 128
_B = BATCH * HIST

_info = plsc.get_sparse_core_info()
_NC = _info.num_cores
_NS = _info.num_subcores
_NW = _NC * _NS
_BPW = _B // _NW
_CH = 64
_NCH = _BPW // _CH
_NBUF = 4
_ROUNDS = _NCH // _NBUF
_UNROLL = 8


def _emb_body(idx_hbm, table_hbm, out_hbm, idx_v, r128_v, r64_v, *sems):
    gsems = sems[:_NBUF]
    osems = sems[_NBUF:]
    wid = lax.axis_index("s") * _NC + lax.axis_index("c")
    base = wid * _BPW
    pltpu.sync_copy(idx_hbm.at[wid], idx_v)

    def g_start(j, b):
        pltpu.async_copy(table_hbm.at[idx_v.at[j]], r128_v.at[b], gsems[b])

    def g_wait(b):
        pltpu.make_async_copy(
            table_hbm.at[idx_v.at[0]], r128_v.at[b], gsems[b]).wait()

    def o_start(j, b):
        pltpu.async_copy(
            r64_v.at[b], out_hbm.at[pl.ds(base + j * _CH, _CH)], osems[b])

    def o_wait(b):
        pltpu.make_async_copy(
            r64_v.at[b], out_hbm.at[pl.ds(base, _CH)], osems[b]).wait()

    def compact(b):
        def crow(r, carry):
            for u in range(_UNROLL):
                rr = r * _UNROLL + u
                for c in range(D_MODEL // 16):
                    r64_v[b, rr, pl.ds(c * 16, 16)] = (
                        r128_v[b, rr, pl.ds(c * 16, 16)])
            return carry
        lax.fori_loop(0, _CH // _UNROLL, crow, 0)

    for b in range(_NBUF):
        g_start(b, b)

    for b in range(_NBUF):
        g_wait(b)
        compact(b)
        o_start(b, b)
        g_start(_NBUF + b, b)

    def round_body(r, carry):
        jbase = r * _NBUF
        for b in range(_NBUF):
            g_wait(b)
            o_wait(b)
            compact(b)
            o_start(jbase + b, b)
            g_start(jbase + _NBUF + b, b)
        return carry

    lax.fori_loop(1, _ROUNDS - 1, round_body, 0)

    jlast = (_ROUNDS - 1) * _NBUF
    for b in range(_NBUF):
        g_wait(b)
        o_wait(b)
        compact(b)
        o_start(jlast + b, b)
    for b in range(_NBUF):
        o_wait(b)


@jax.jit
def _embed(idx3d, table128):
    mesh = plsc.VectorSubcoreMesh(core_axis_name="c", subcore_axis_name="s")
    k = functools.partial(
        pl.kernel,
        mesh=mesh,
        out_type=jax.ShapeDtypeStruct((_B, D_MODEL), jnp.float32),
        scratch_types=[
            pltpu.VMEM((_NCH, _CH), jnp.int32),
            pltpu.VMEM((_NBUF, _CH, _DPAD), jnp.float32),
            pltpu.VMEM((_NBUF, _CH, D_MODEL), jnp.float32),
        ] + [pltpu.SemaphoreType.DMA] * (2 * _NBUF),
    )(_emb_body)
    return k(idx3d, table128)


def kernel(input_ids, embedding_weight):
    idx = input_ids.reshape(-1).astype(jnp.int32)
    idx3d = idx.reshape(_NW, _NCH, _CH)
    table128 = jnp.pad(embedding_weight, ((0, 0), (0, _DPAD - D_MODEL)))
    out = _embed(idx3d, table128)
    return out.reshape(BATCH, HIST, D_MODEL)
